# Initial kernel scaffold; baseline (speedup 1.0000x reference)
import jax
import jax.numpy as jnp
from jax.experimental import pallas as pl


def kernel(x, weight):
    raise NotImplementedError("write your pallas kernel here")


# speedup vs baseline: 1.0223x; 1.0223x over previous
import functools

import jax
import jax.numpy as jnp
from jax import lax
from jax.experimental import pallas as pl
from jax.experimental.pallas import tpu as pltpu
from jax.experimental.pallas import tpu_sc as plsc

BATCH = 16384
HIST = 50
DIM = 32
TOTAL = BATCH * HIST
NUM_WORKERS = 32
PER_WORKER = TOTAL // NUM_WORKERS
CHUNK = 128
STEPS = PER_WORKER // CHUNK

_mesh = plsc.VectorSubcoreMesh(core_axis_name="c", subcore_axis_name="s")


@functools.partial(
    pl.kernel,
    mesh=_mesh,
    out_type=jax.ShapeDtypeStruct((TOTAL, DIM), jnp.float32),
    scratch_types=[
        pltpu.VMEM((STEPS, CHUNK), jnp.int32),
        pltpu.VMEM((CHUNK, DIM), jnp.float32),
        pltpu.SemaphoreType.DMA,
    ],
    compiler_params=pltpu.CompilerParams(use_tc_tiling_on_sc=False),
)
def _sc_gather(idx_hbm, table_hbm, out_hbm, idx_v, rows_v, sem):
    wid = lax.axis_index("s") * 2 + lax.axis_index("c")
    pltpu.sync_copy(idx_hbm.at[wid], idx_v)
    base = wid * PER_WORKER

    def body(g, carry):
        pltpu.async_copy(table_hbm.at[idx_v.at[g]], rows_v, sem).wait()
        pltpu.sync_copy(rows_v, out_hbm.at[pl.ds(base + g * CHUNK, CHUNK)])
        return carry

    lax.fori_loop(0, STEPS, body, 0)


def kernel(x, weight):
    idx = x.reshape(NUM_WORKERS, STEPS, CHUNK)
    out = _sc_gather(idx, weight)
    return out.reshape(BATCH, HIST, DIM)


# speedup vs baseline: 1.1133x; 1.0890x over previous
import functools

import jax
import jax.numpy as jnp
from jax import lax
from jax.experimental import pallas as pl
from jax.experimental.pallas import tpu as pltpu
from jax.experimental.pallas import tpu_sc as plsc

BATCH = 16384
HIST = 50
DIM = 32
TOTAL = BATCH * HIST
NUM_WORKERS = 32
PER_WORKER = TOTAL // NUM_WORKERS
CHUNK = 128
STEPS = PER_WORKER // CHUNK
K = 10
NSUP = STEPS // K
ROWS = K * CHUNK

_mesh = plsc.VectorSubcoreMesh(core_axis_name="c", subcore_axis_name="s")


@functools.partial(
    pl.kernel,
    mesh=_mesh,
    out_type=jax.ShapeDtypeStruct((TOTAL, DIM), jnp.float32),
    scratch_types=[
        pltpu.VMEM((STEPS, CHUNK), jnp.int32),
        pltpu.VMEM((2, ROWS, DIM), jnp.float32),
        pltpu.SemaphoreType.DMA,
        pltpu.SemaphoreType.DMA,
        pltpu.SemaphoreType.DMA,
        pltpu.SemaphoreType.DMA,
    ],
    compiler_params=pltpu.CompilerParams(use_tc_tiling_on_sc=False),
)
def _sc_gather(idx_hbm, table_hbm, out_hbm, idx_v, rows_v, g0, g1, o0, o1):
    wid = lax.axis_index("s") * 2 + lax.axis_index("c")
    pltpu.sync_copy(idx_hbm.at[wid], idx_v)
    base = wid * PER_WORKER
    gsem = (g0, g1)
    osem = (o0, o1)

    def fire_gathers(s, b):
        for j in range(K):
            pltpu.async_copy(
                table_hbm.at[idx_v.at[s * K + j]],
                rows_v.at[b].at[pl.ds(j * CHUNK, CHUNK)],
                gsem[b],
            )

    def drain_gathers(b):
        pltpu.make_async_copy(
            table_hbm.at[pl.ds(0, ROWS)], rows_v.at[b], gsem[b]
        ).wait()

    def drain_out(b):
        pltpu.make_async_copy(
            rows_v.at[b], out_hbm.at[pl.ds(0, ROWS)], osem[b]
        ).wait()

    def step(t, b):
        nb = 1 - b

        @pl.when(t >= 1)
        def _():
            drain_out(nb)

        @pl.when(t + 1 < NSUP)
        def _():
            fire_gathers(t + 1, nb)

        drain_gathers(b)
        pltpu.async_copy(
            rows_v.at[b],
            out_hbm.at[pl.ds(base + t * ROWS, ROWS)],
            osem[b],
        )

    fire_gathers(0, 0)

    def body(t2, carry):
        step(2 * t2, 0)
        step(2 * t2 + 1, 1)
        return carry

    lax.fori_loop(0, NSUP // 2, body, 0)
    drain_out(1)


def kernel(x, weight):
    idx = x.reshape(NUM_WORKERS, STEPS, CHUNK)
    out = _sc_gather(idx, weight)
    return out.reshape(BATCH, HIST, DIM)


# speedup vs baseline: 1.9406x; 1.7431x over previous
import functools

import jax
import jax.numpy as jnp
from jax import lax
from jax.experimental import pallas as pl
from jax.experimental.pallas import tpu as pltpu
from jax.experimental.pallas import tpu_sc as plsc

BATCH = 16384
HIST = 50
DIM = 32
TOTAL = BATCH * HIST
NUM_WORKERS = 32
PER_WORKER = TOTAL // NUM_WORKERS
CHUNK = 128
STEPS = PER_WORKER // CHUNK
K = 10
NSUP = STEPS // K
ROWS = K * CHUNK

_mesh = plsc.VectorSubcoreMesh(core_axis_name="c", subcore_axis_name="s")


@functools.partial(
    pl.kernel,
    mesh=_mesh,
    out_type=jax.ShapeDtypeStruct((TOTAL, DIM), jnp.float32),
    scratch_types=[
        pltpu.VMEM((STEPS, CHUNK), jnp.int32),
        pltpu.VMEM((2, ROWS, DIM), jnp.float32),
        pltpu.SemaphoreType.DMA,
        pltpu.SemaphoreType.DMA,
        pltpu.SemaphoreType.DMA,
        pltpu.SemaphoreType.DMA,
    ],
    compiler_params=pltpu.CompilerParams(use_tc_tiling_on_sc=False),
)
def _sc_gather(idx_hbm, table_hbm, out_hbm, idx_v, rows_v, g0, g1, o0, o1):
    wid = lax.axis_index("s") * 2 + lax.axis_index("c")
    pltpu.sync_copy(idx_hbm.at[wid], idx_v)
    base = wid * PER_WORKER
    gsem = (g0, g1)
    osem = (o0, o1)

    def fire_gathers(s, b):
        for j in range(K):
            pltpu.async_copy(
                table_hbm.at[idx_v.at[s * K + j]],
                rows_v.at[b].at[pl.ds(j * CHUNK, CHUNK)],
                gsem[b],
            )

    def drain_gathers(b):
        pltpu.make_async_copy(
            table_hbm.at[pl.ds(0, ROWS)], rows_v.at[b], gsem[b]
        ).wait()

    def drain_out(b):
        pltpu.make_async_copy(
            rows_v.at[b], out_hbm.at[pl.ds(0, ROWS)], osem[b]
        ).wait()

    def step(t, b):
        nb = 1 - b

        @pl.when(t >= 1)
        def _():
            drain_out(nb)

        @pl.when(t + 1 < NSUP)
        def _():
            fire_gathers(t + 1, nb)

        drain_gathers(b)
        pltpu.async_copy(
            rows_v.at[b],
            out_hbm.at[pl.ds(base + t * ROWS, ROWS)],
            osem[b],
        )

    fire_gathers(0, 0)

    def body(t2, carry):
        step(2 * t2, 0)
        step(2 * t2 + 1, 1)
        return carry

    lax.fori_loop(0, NSUP // 2, body, 0)
    drain_out(1)


def kernel(x, weight):
    idx = x.T.reshape(NUM_WORKERS, STEPS, CHUNK)
    out = _sc_gather(idx, weight)
    return out.reshape(HIST, BATCH, DIM).transpose(1, 0, 2)


# speedup vs baseline: 2.7863x; 1.4358x over previous
import functools

import jax
import jax.numpy as jnp
from jax import lax
from jax.experimental import pallas as pl
from jax.experimental.pallas import tpu as pltpu
from jax.experimental.pallas import tpu_sc as plsc

BATCH = 16384
HIST = 50
DIM = 32
TOTAL = BATCH * HIST
NUM_WORKERS = 32
PER_WORKER = TOTAL // NUM_WORKERS
CHUNK = 128
STEPS = PER_WORKER // CHUNK
LANES = 16
BB = BATCH // CHUNK

_mesh = plsc.VectorSubcoreMesh(core_axis_name="c", subcore_axis_name="s")


@functools.partial(
    pl.kernel,
    mesh=_mesh,
    out_type=jax.ShapeDtypeStruct((HIST, DIM // 8, BB, 8, CHUNK), jnp.float32),
    scratch_types=[
        pltpu.VMEM((STEPS, CHUNK), jnp.int32),
        pltpu.VMEM((2, CHUNK, DIM), jnp.float32),
        pltpu.VMEM((2, DIM // 8, 8, CHUNK + 1), jnp.float32),
        pltpu.SemaphoreType.DMA,
        pltpu.SemaphoreType.DMA,
        pltpu.SemaphoreType.DMA,
        pltpu.SemaphoreType.DMA,
    ],
    compiler_params=pltpu.CompilerParams(
        use_tc_tiling_on_sc=False, needs_layout_passes=False
    ),
)
def _sc_gather(idx_hbm, table_hbm, out_hbm, idx_v, rows_v, t_v, g0, g1, o0, o1):
    wid = lax.axis_index("s") * 2 + lax.axis_index("c")
    pltpu.sync_copy(idx_hbm.at[wid], idx_v)
    cbase = wid * STEPS
    gsem = (g0, g1)
    osem = (o0, o1)

    iota = lax.iota(jnp.int32, LANES)
    zeros16 = jnp.zeros((LANES,), jnp.int32)
    db0 = iota // 8
    s0 = iota % 8
    db1 = (iota + LANES) // 8
    s1 = (iota + LANES) % 8

    def fire_gather(t, b):
        pltpu.async_copy(table_hbm.at[idx_v.at[t]], rows_v.at[b], gsem[b])

    def drain_gather(b):
        pltpu.make_async_copy(
            table_hbm.at[pl.ds(0, CHUNK)], rows_v.at[b], gsem[b]
        ).wait()

    def t_src(b):
        return t_v.at[b].at[:, :, pl.ds(0, CHUNK)]

    def drain_out(b):
        pltpu.make_async_copy(
            t_src(b), out_hbm.at[0, :, 0], osem[b]
        ).wait()

    def step(t, b):
        nb = 1 - b

        @pl.when(t + 1 < STEPS)
        def _():
            fire_gather(t + 1, nb)

        drain_gather(b)

        @pl.when(t >= 2)
        def _():
            drain_out(b)

        rv = rows_v.at[b]
        tv = t_v.at[b]

        def trans_r(r, carry):
            lvec = zeros16 + r
            v0 = rv[r, pl.ds(0, LANES)]
            v1 = rv[r, pl.ds(LANES, LANES)]
            plsc.store_scatter(tv, [db0, s0, lvec], v0)
            plsc.store_scatter(tv, [db1, s1, lvec], v1)
            return carry

        lax.fori_loop(0, CHUNK, trans_r, 0)

        c = cbase + t
        h = c // BB
        bb = c % BB
        pltpu.async_copy(t_src(b), out_hbm.at[h, :, bb], osem[b])

    fire_gather(0, 0)

    def body(t2, carry):
        step(2 * t2, 0)
        step(2 * t2 + 1, 1)
        return carry

    lax.fori_loop(0, STEPS // 2, body, 0)
    drain_out(0)
    drain_out(1)


def kernel(x, weight):
    idx = x.T.reshape(NUM_WORKERS, STEPS, CHUNK)
    out5 = _sc_gather(idx, weight)
    t = out5.transpose(0, 1, 3, 2, 4)
    return t.reshape(HIST, DIM, BATCH).transpose(2, 0, 1)
